# baseline (speedup 1.0000x reference)
import functools

import jax
import jax.numpy as jnp
from jax import lax
from jax.experimental import pallas as pl
from jax.experimental.pallas import tpu as pltpu
from jax.experimental.pallas import tpu_sc as plsc

BATCH = 4096
SEQ = 200
VOCAB = 1000000
EMBED = 64
HIDDEN = 128
NUM_OUT = 3

NC = 2
NS = 16
NW = NC * NS
BPW = BATCH // NW
C0, C1 = 120, 80
LANES = 16
PACKED = EMBED // 2
NTC_FULL = VOCAB // 128
TPW = NTC_FULL // NW
TAIL_TC = NTC_FULL
REM_FULL = NTC_FULL - NW * TPW


def _sc_transpose_pack(embT, tail):
    mesh = plsc.VectorSubcoreMesh(core_axis_name="c", subcore_axis_name="s")

    @functools.partial(
        pl.kernel,
        out_type=jax.ShapeDtypeStruct((VOCAB * PACKED,), jnp.int32),
        mesh=mesh,
        compiler_params=pltpu.CompilerParams(
            use_tc_tiling_on_sc=True, needs_layout_passes=False),
        scratch_types=[
            pltpu.VMEM((EMBED, 128), jnp.float32),
            pltpu.VMEM((EMBED, 128), jnp.float32),
            pltpu.VMEM((EMBED, 64), jnp.float32),
            pltpu.VMEM((128 * PACKED,), jnp.int32),
            pltpu.VMEM((128 * PACKED,), jnp.int32),
            pltpu.SemaphoreType.DMA,
            pltpu.SemaphoreType.DMA,
            pltpu.SemaphoreType.DMA,
            pltpu.SemaphoreType.DMA,
        ],
    )
    def k(in_hbm, tail_hbm, out_hbm, w0, w1, wt, ob0, ob1,
          si0, si1, so0, so1):
        wid = lax.axis_index("s") * NC + lax.axis_index("c")
        t0 = wid * TPW

        j32 = lax.iota(jnp.int32, LANES) * PACKED

        def start_in(t, win, sem):
            col = pl.multiple_of(t * 128, 128)
            pltpu.make_async_copy(
                in_hbm.at[:, pl.ds(col, 128)], win, sem).start()

        def drain_in(win, sem):
            pltpu.make_async_copy(
                in_hbm.at[:, pl.ds(0, 128)], win, sem).wait()

        def transpose(win, ob, njblk):
            for jb in range(njblk):
                for dp in range(PACKED):
                    a = win[2 * dp, pl.ds(jb * LANES, LANES)]
                    b = win[2 * dp + 1, pl.ds(jb * LANES, LANES)]
                    v = plsc.bitcast(
                        plsc.pack(a, b, format=plsc.PackFormat.INTERLEAVED),
                        jnp.int32)
                    plsc.store_scatter(
                        ob, [j32 + (jb * LANES * PACKED + dp)], v)

        def start_out(t, ob, sem):
            pltpu.make_async_copy(
                ob, out_hbm.at[pl.ds(t * (128 * PACKED), 128 * PACKED)],
                sem).start()

        def drain_out(ob, sem):
            pltpu.make_async_copy(
                ob, out_hbm.at[pl.ds(0, 128 * PACKED)], sem).wait()

        start_in(t0, w0, si0)

        @pl.loop(0, TPW, step=2)
        def _(i):
            t = t0 + i
            start_in(t + 1, w1, si1)
            drain_in(w0, si0)

            @pl.when(i > 0)
            def _():
                drain_out(ob0, so0)

            transpose(w0, ob0, 8)
            start_out(t, ob0, so0)

            @pl.when(i + 2 < TPW)
            def _():
                start_in(t + 2, w0, si0)

            drain_in(w1, si1)

            @pl.when(i > 0)
            def _():
                drain_out(ob1, so1)

            transpose(w1, ob1, 8)
            start_out(t + 1, ob1, so1)

        drain_out(ob0, so0)
        drain_out(ob1, so1)

        @pl.when(wid < REM_FULL)
        def _():
            t = NW * TPW + wid
            start_in(t, w0, si0)
            drain_in(w0, si0)
            transpose(w0, ob0, 8)
            start_out(t, ob0, so0)
            drain_out(ob0, so0)

        @pl.when(wid == REM_FULL)
        def _():
            c = pltpu.make_async_copy(tail_hbm, wt, si1)
            c.start()
            c.wait()
            transpose(wt, ob1, 4)
            co = pltpu.make_async_copy(
                ob1.at[pl.ds(0, 64 * PACKED)],
                out_hbm.at[pl.ds(TAIL_TC * (128 * PACKED), 64 * PACKED)],
                so1)
            co.start()
            co.wait()

    return k(embT, tail)


def _sc_gather_pool(flat_ids, table_p):
    mesh = plsc.VectorSubcoreMesh(core_axis_name="c", subcore_axis_name="s")

    @functools.partial(
        pl.kernel,
        out_type=jax.ShapeDtypeStruct((BATCH, EMBED), jnp.float32),
        mesh=mesh,
        compiler_params=pltpu.CompilerParams(
            use_tc_tiling_on_sc=False, needs_layout_passes=False),
        scratch_types=[
            pltpu.VMEM((BPW * SEQ,), jnp.int32),
            pltpu.VMEM((C0, PACKED), jnp.int32),
            pltpu.VMEM((C1, PACKED), jnp.int32),
            pltpu.VMEM((C0, PACKED), jnp.int32),
            pltpu.VMEM((C1, PACKED), jnp.int32),
            pltpu.VMEM((BPW, EMBED), jnp.float32),
            pltpu.SemaphoreType.DMA,
            pltpu.SemaphoreType.DMA,
        ],
    )
    def k(ids_hbm, tbl_hbm, out_hbm, idx_v, a0, a1, b0, b1, pooled_v,
          sem_a, sem_b):
        wid = lax.axis_index("s") * NC + lax.axis_index("c")
        base = wid * BPW
        pltpu.sync_copy(ids_hbm.at[pl.ds(base * SEQ, BPW * SEQ)], idx_v)

        def start(b, r0, r1, sem):
            off = b * SEQ
            pltpu.make_async_copy(
                tbl_hbm.at[idx_v.at[pl.ds(off, C0)]], r0, sem).start()
            pltpu.make_async_copy(
                tbl_hbm.at[idx_v.at[pl.ds(off + C0, C1)]], r1, sem).start()

        def drain(r0, r1, sem):
            pltpu.make_async_copy(
                tbl_hbm.at[idx_v.at[pl.ds(0, C0)]], r0, sem).wait()
            pltpu.make_async_copy(
                tbl_hbm.at[idx_v.at[pl.ds(0, C1)]], r1, sem).wait()

        def row_add(ref, r, acc):
            new = []
            for h in range(2):
                w = plsc.bitcast(
                    ref[r, pl.ds(h * LANES, LANES)], jnp.bfloat16)
                lo, hi = plsc.unpack(w, format=plsc.PackFormat.INTERLEAVED)
                new.append(acc[2 * h] + lo)
                new.append(acc[2 * h + 1] + hi)
            return tuple(new)

        def accumulate(b, r0, r1):
            acc = lax.fori_loop(
                0, C0, lambda r, acc: row_add(r0, r, acc),
                tuple(jnp.zeros((LANES,), jnp.float32) for _ in range(4)),
            )
            acc = lax.fori_loop(
                0, C1, lambda r, acc: row_add(r1, r, acc), acc)

            evens = lax.iota(jnp.int32, LANES) * 2
            for h in range(2):
                colbase = h * 2 * LANES
                plsc.store_scatter(
                    pooled_v.at[b], [colbase + evens], acc[2 * h])
                plsc.store_scatter(
                    pooled_v.at[b], [colbase + evens + 1], acc[2 * h + 1])

        start(0, a0, a1, sem_a)

        @pl.loop(0, BPW, step=2)
        def _(b):
            start(b + 1, b0, b1, sem_b)
            drain(a0, a1, sem_a)
            accumulate(b, a0, a1)

            @pl.when(b + 2 < BPW)
            def _():
                start(b + 2, a0, a1, sem_a)

            drain(b0, b1, sem_b)
            accumulate(b + 1, b0, b1)

        pltpu.sync_copy(pooled_v, out_hbm.at[pl.ds(base, BPW)])

    return k(flat_ids, table_p)


def _mlp(pooled, W1, b1, W2, b2):
    BB = 512

    def body(p_ref, w1_ref, b1_ref, w2_ref, b2_ref, o_ref):
        x = p_ref[...] * (1.0 / SEQ)
        h = jnp.dot(x, w1_ref[...], preferred_element_type=jnp.float32)
        h = jnp.maximum(h + b1_ref[...], 0.0)
        o_ref[...] = (
            jnp.dot(h, w2_ref[...], preferred_element_type=jnp.float32)
            + b2_ref[...]
        )

    return pl.pallas_call(
        body,
        grid=(BATCH // BB,),
        in_specs=[
            pl.BlockSpec((BB, EMBED), lambda i: (i, 0)),
            pl.BlockSpec((EMBED, HIDDEN), lambda i: (0, 0)),
            pl.BlockSpec((1, HIDDEN), lambda i: (0, 0)),
            pl.BlockSpec((HIDDEN, NUM_OUT), lambda i: (0, 0)),
            pl.BlockSpec((1, NUM_OUT), lambda i: (0, 0)),
        ],
        out_specs=pl.BlockSpec((BB, NUM_OUT), lambda i: (i, 0)),
        out_shape=jax.ShapeDtypeStruct((BATCH, NUM_OUT), jnp.float32),
    )(pooled, W1, b1.reshape(1, HIDDEN), W2, b2.reshape(1, NUM_OUT))


def kernel(input_ids, embedding, W1, b1, W2, b2):
    flat_ids = input_ids.reshape(-1).astype(jnp.int32)
    embT = embedding.T
    packed = _sc_transpose_pack(embT, embT[:, NTC_FULL * 128:])
    table_p = packed.reshape(VOCAB, PACKED)
    pooled = _sc_gather_pool(flat_ids, table_p)
    return _mlp(pooled, W1, b1, W2, b2)


# speedup vs baseline: 1.4018x; 1.4018x over previous
import functools

import jax
import jax.numpy as jnp
from jax import lax
from jax.experimental import pallas as pl
from jax.experimental.pallas import tpu as pltpu
from jax.experimental.pallas import tpu_sc as plsc

BATCH = 4096
SEQ = 200
VOCAB = 1000000
EMBED = 64
HIDDEN = 128
NUM_OUT = 3

NC = 2
NS = 16
NW = NC * NS
BPW = BATCH // NW
C0, C1 = 120, 80
LANES = 16
PACKED = EMBED // 2
NTC_FULL = VOCAB // 128
TPW = NTC_FULL // NW
TAIL_TC = NTC_FULL
REM_FULL = NTC_FULL - NW * TPW


def _sc_transpose_pack(embT, tail):
    mesh = plsc.VectorSubcoreMesh(core_axis_name="c", subcore_axis_name="s")

    @functools.partial(
        pl.kernel,
        out_type=jax.ShapeDtypeStruct((VOCAB * PACKED,), jnp.int32),
        mesh=mesh,
        compiler_params=pltpu.CompilerParams(
            use_tc_tiling_on_sc=True, needs_layout_passes=False),
        scratch_types=[
            pltpu.VMEM((EMBED, 128), jnp.float32),
            pltpu.VMEM((EMBED, 128), jnp.float32),
            pltpu.VMEM((EMBED, 64), jnp.float32),
            pltpu.VMEM((128 * PACKED,), jnp.int32),
            pltpu.VMEM((128 * PACKED,), jnp.int32),
            pltpu.VMEM((LANES * (LANES + 1),), jnp.int32),
            pltpu.SemaphoreType.DMA,
            pltpu.SemaphoreType.DMA,
            pltpu.SemaphoreType.DMA,
            pltpu.SemaphoreType.DMA,
        ],
    )
    def k(in_hbm, tail_hbm, out_hbm, w0, w1, wt, ob0, ob1, stage,
          si0, si1, so0, so1):
        wid = lax.axis_index("s") * NC + lax.axis_index("c")
        t0 = wid * TPW

        jskew = lax.iota(jnp.int32, LANES) * (LANES + 1)

        def start_in(t, win, sem):
            col = pl.multiple_of(t * 128, 128)
            pltpu.make_async_copy(
                in_hbm.at[:, pl.ds(col, 128)], win, sem).start()

        def drain_in(win, sem):
            pltpu.make_async_copy(
                in_hbm.at[:, pl.ds(0, 128)], win, sem).wait()

        def transpose(win, ob, njblk):
            for jb in range(njblk):
                for h in range(2):
                    for dp2 in range(LANES):
                        dp = h * LANES + dp2
                        a = win[2 * dp, pl.ds(jb * LANES, LANES)]
                        b = win[2 * dp + 1, pl.ds(jb * LANES, LANES)]
                        v = plsc.bitcast(
                            plsc.pack(
                                a, b, format=plsc.PackFormat.INTERLEAVED),
                            jnp.int32)
                        plsc.store_scatter(stage, [jskew + dp2], v)
                    for j2 in range(LANES):
                        ob[pl.ds((jb * LANES + j2) * PACKED + h * LANES,
                                 LANES)] = stage[pl.ds(j2 * (LANES + 1),
                                                       LANES)]

        def start_out(t, ob, sem):
            pltpu.make_async_copy(
                ob, out_hbm.at[pl.ds(t * (128 * PACKED), 128 * PACKED)],
                sem).start()

        def drain_out(ob, sem):
            pltpu.make_async_copy(
                ob, out_hbm.at[pl.ds(0, 128 * PACKED)], sem).wait()

        start_in(t0, w0, si0)

        @pl.loop(0, TPW, step=2)
        def _(i):
            t = t0 + i
            start_in(t + 1, w1, si1)
            drain_in(w0, si0)

            @pl.when(i > 0)
            def _():
                drain_out(ob0, so0)

            transpose(w0, ob0, 8)
            start_out(t, ob0, so0)

            @pl.when(i + 2 < TPW)
            def _():
                start_in(t + 2, w0, si0)

            drain_in(w1, si1)

            @pl.when(i > 0)
            def _():
                drain_out(ob1, so1)

            transpose(w1, ob1, 8)
            start_out(t + 1, ob1, so1)

        drain_out(ob0, so0)
        drain_out(ob1, so1)

        @pl.when(wid < REM_FULL)
        def _():
            t = NW * TPW + wid
            start_in(t, w0, si0)
            drain_in(w0, si0)
            transpose(w0, ob0, 8)
            start_out(t, ob0, so0)
            drain_out(ob0, so0)

        @pl.when(wid == REM_FULL)
        def _():
            c = pltpu.make_async_copy(tail_hbm, wt, si1)
            c.start()
            c.wait()
            transpose(wt, ob1, 4)
            co = pltpu.make_async_copy(
                ob1.at[pl.ds(0, 64 * PACKED)],
                out_hbm.at[pl.ds(TAIL_TC * (128 * PACKED), 64 * PACKED)],
                so1)
            co.start()
            co.wait()

    return k(embT, tail)


def _sc_gather_pool(flat_ids, table_p):
    mesh = plsc.VectorSubcoreMesh(core_axis_name="c", subcore_axis_name="s")

    @functools.partial(
        pl.kernel,
        out_type=jax.ShapeDtypeStruct((BATCH, EMBED), jnp.float32),
        mesh=mesh,
        compiler_params=pltpu.CompilerParams(
            use_tc_tiling_on_sc=False, needs_layout_passes=False),
        scratch_types=[
            pltpu.VMEM((BPW * SEQ,), jnp.int32),
            pltpu.VMEM((C0, PACKED), jnp.int32),
            pltpu.VMEM((C1, PACKED), jnp.int32),
            pltpu.VMEM((C0, PACKED), jnp.int32),
            pltpu.VMEM((C1, PACKED), jnp.int32),
            pltpu.VMEM((BPW, EMBED), jnp.float32),
            pltpu.SemaphoreType.DMA,
            pltpu.SemaphoreType.DMA,
        ],
    )
    def k(ids_hbm, tbl_hbm, out_hbm, idx_v, a0, a1, b0, b1, pooled_v,
          sem_a, sem_b):
        wid = lax.axis_index("s") * NC + lax.axis_index("c")
        base = wid * BPW
        pltpu.sync_copy(ids_hbm.at[pl.ds(base * SEQ, BPW * SEQ)], idx_v)

        def start(b, r0, r1, sem):
            off = b * SEQ
            pltpu.make_async_copy(
                tbl_hbm.at[idx_v.at[pl.ds(off, C0)]], r0, sem).start()
            pltpu.make_async_copy(
                tbl_hbm.at[idx_v.at[pl.ds(off + C0, C1)]], r1, sem).start()

        def drain(r0, r1, sem):
            pltpu.make_async_copy(
                tbl_hbm.at[idx_v.at[pl.ds(0, C0)]], r0, sem).wait()
            pltpu.make_async_copy(
                tbl_hbm.at[idx_v.at[pl.ds(0, C1)]], r1, sem).wait()

        def row_add(ref, r, acc):
            new = []
            for h in range(2):
                w = plsc.bitcast(
                    ref[r, pl.ds(h * LANES, LANES)], jnp.bfloat16)
                lo, hi = plsc.unpack(w, format=plsc.PackFormat.INTERLEAVED)
                new.append(acc[2 * h] + lo)
                new.append(acc[2 * h + 1] + hi)
            return tuple(new)

        def accumulate(b, r0, r1):
            acc = lax.fori_loop(
                0, C0, lambda r, acc: row_add(r0, r, acc),
                tuple(jnp.zeros((LANES,), jnp.float32) for _ in range(4)),
            )
            acc = lax.fori_loop(
                0, C1, lambda r, acc: row_add(r1, r, acc), acc)

            evens = lax.iota(jnp.int32, LANES) * 2
            for h in range(2):
                colbase = h * 2 * LANES
                plsc.store_scatter(
                    pooled_v.at[b], [colbase + evens], acc[2 * h])
                plsc.store_scatter(
                    pooled_v.at[b], [colbase + evens + 1], acc[2 * h + 1])

        start(0, a0, a1, sem_a)

        @pl.loop(0, BPW, step=2)
        def _(b):
            start(b + 1, b0, b1, sem_b)
            drain(a0, a1, sem_a)
            accumulate(b, a0, a1)

            @pl.when(b + 2 < BPW)
            def _():
                start(b + 2, a0, a1, sem_a)

            drain(b0, b1, sem_b)
            accumulate(b + 1, b0, b1)

        pltpu.sync_copy(pooled_v, out_hbm.at[pl.ds(base, BPW)])

    return k(flat_ids, table_p)


def _mlp(pooled, W1, b1, W2, b2):
    BB = 512

    def body(p_ref, w1_ref, b1_ref, w2_ref, b2_ref, o_ref):
        x = p_ref[...] * (1.0 / SEQ)
        h = jnp.dot(x, w1_ref[...], preferred_element_type=jnp.float32)
        h = jnp.maximum(h + b1_ref[...], 0.0)
        o_ref[...] = (
            jnp.dot(h, w2_ref[...], preferred_element_type=jnp.float32)
            + b2_ref[...]
        )

    return pl.pallas_call(
        body,
        grid=(BATCH // BB,),
        in_specs=[
            pl.BlockSpec((BB, EMBED), lambda i: (i, 0)),
            pl.BlockSpec((EMBED, HIDDEN), lambda i: (0, 0)),
            pl.BlockSpec((1, HIDDEN), lambda i: (0, 0)),
            pl.BlockSpec((HIDDEN, NUM_OUT), lambda i: (0, 0)),
            pl.BlockSpec((1, NUM_OUT), lambda i: (0, 0)),
        ],
        out_specs=pl.BlockSpec((BB, NUM_OUT), lambda i: (i, 0)),
        out_shape=jax.ShapeDtypeStruct((BATCH, NUM_OUT), jnp.float32),
    )(pooled, W1, b1.reshape(1, HIDDEN), W2, b2.reshape(1, NUM_OUT))


def kernel(input_ids, embedding, W1, b1, W2, b2):
    flat_ids = input_ids.reshape(-1).astype(jnp.int32)
    embT = embedding.T
    packed = _sc_transpose_pack(embT, embT[:, NTC_FULL * 128:])
    table_p = packed.reshape(VOCAB, PACKED)
    pooled = _sc_gather_pool(flat_ids, table_p)
    return _mlp(pooled, W1, b1, W2, b2)


# speedup vs baseline: 1.4534x; 1.0368x over previous
import functools

import jax
import jax.numpy as jnp
from jax import lax
from jax.experimental import pallas as pl
from jax.experimental.pallas import tpu as pltpu
from jax.experimental.pallas import tpu_sc as plsc

BATCH = 4096
SEQ = 200
VOCAB = 1000000
EMBED = 64
HIDDEN = 128
NUM_OUT = 3

NC = 2
NS = 16
NW = NC * NS
BPW = BATCH // NW
C0, C1 = 120, 80
LANES = 16
PACKED = EMBED // 2
NTC_FULL = VOCAB // 128
TPW = NTC_FULL // NW
TAIL_TC = NTC_FULL
REM_FULL = NTC_FULL - NW * TPW


def _sc_transpose_pack(embT, tail):
    mesh = plsc.VectorSubcoreMesh(core_axis_name="c", subcore_axis_name="s")

    @functools.partial(
        pl.kernel,
        out_type=jax.ShapeDtypeStruct((VOCAB * PACKED,), jnp.int32),
        mesh=mesh,
        compiler_params=pltpu.CompilerParams(
            use_tc_tiling_on_sc=True, needs_layout_passes=False),
        scratch_types=[
            pltpu.VMEM((EMBED, 128), jnp.float32),
            pltpu.VMEM((EMBED, 128), jnp.float32),
            pltpu.VMEM((EMBED, 64), jnp.float32),
            pltpu.VMEM((128 * PACKED,), jnp.int32),
            pltpu.VMEM((128 * PACKED,), jnp.int32),
            pltpu.VMEM((LANES * (LANES + 1),), jnp.int32),
            pltpu.VMEM((LANES * (LANES + 1),), jnp.int32),
            pltpu.SemaphoreType.DMA,
            pltpu.SemaphoreType.DMA,
            pltpu.SemaphoreType.DMA,
            pltpu.SemaphoreType.DMA,
        ],
    )
    def k(in_hbm, tail_hbm, out_hbm, w0, w1, wt, ob0, ob1, stage0, stage1,
          si0, si1, so0, so1):
        wid = lax.axis_index("s") * NC + lax.axis_index("c")
        t0 = wid * TPW

        jskew = lax.iota(jnp.int32, LANES) * (LANES + 1)

        def start_in(t, win, sem):
            col = pl.multiple_of(t * 128, 128)
            pltpu.make_async_copy(
                in_hbm.at[:, pl.ds(col, 128)], win, sem).start()

        def drain_in(win, sem):
            pltpu.make_async_copy(
                in_hbm.at[:, pl.ds(0, 128)], win, sem).wait()

        def transpose(win, ob, njblk):
            for jb in range(njblk):
                for h in range(2):
                    stg = stage0 if h == 0 else stage1
                    for dp2 in range(LANES):
                        dp = h * LANES + dp2
                        a = win[2 * dp, pl.ds(jb * LANES, LANES)]
                        b = win[2 * dp + 1, pl.ds(jb * LANES, LANES)]
                        v = plsc.bitcast(
                            plsc.pack(
                                a, b, format=plsc.PackFormat.INTERLEAVED),
                            jnp.int32)
                        plsc.store_scatter(stg, [jskew + dp2], v)
                for h in range(2):
                    stg = stage0 if h == 0 else stage1
                    for j2 in range(LANES):
                        ob[pl.ds((jb * LANES + j2) * PACKED + h * LANES,
                                 LANES)] = stg[pl.ds(j2 * (LANES + 1),
                                                     LANES)]

        def start_out(t, ob, sem):
            pltpu.make_async_copy(
                ob, out_hbm.at[pl.ds(t * (128 * PACKED), 128 * PACKED)],
                sem).start()

        def drain_out(ob, sem):
            pltpu.make_async_copy(
                ob, out_hbm.at[pl.ds(0, 128 * PACKED)], sem).wait()

        start_in(t0, w0, si0)

        @pl.loop(0, TPW, step=2)
        def _(i):
            t = t0 + i
            start_in(t + 1, w1, si1)
            drain_in(w0, si0)

            @pl.when(i > 0)
            def _():
                drain_out(ob0, so0)

            transpose(w0, ob0, 8)
            start_out(t, ob0, so0)

            @pl.when(i + 2 < TPW)
            def _():
                start_in(t + 2, w0, si0)

            drain_in(w1, si1)

            @pl.when(i > 0)
            def _():
                drain_out(ob1, so1)

            transpose(w1, ob1, 8)
            start_out(t + 1, ob1, so1)

        drain_out(ob0, so0)
        drain_out(ob1, so1)

        @pl.when(wid < REM_FULL)
        def _():
            t = NW * TPW + wid
            start_in(t, w0, si0)
            drain_in(w0, si0)
            transpose(w0, ob0, 8)
            start_out(t, ob0, so0)
            drain_out(ob0, so0)

        @pl.when(wid == REM_FULL)
        def _():
            c = pltpu.make_async_copy(tail_hbm, wt, si1)
            c.start()
            c.wait()
            transpose(wt, ob1, 4)
            co = pltpu.make_async_copy(
                ob1.at[pl.ds(0, 64 * PACKED)],
                out_hbm.at[pl.ds(TAIL_TC * (128 * PACKED), 64 * PACKED)],
                so1)
            co.start()
            co.wait()

    return k(embT, tail)


def _sc_gather_pool(flat_ids, table_p):
    mesh = plsc.VectorSubcoreMesh(core_axis_name="c", subcore_axis_name="s")

    @functools.partial(
        pl.kernel,
        out_type=jax.ShapeDtypeStruct((BATCH, EMBED), jnp.float32),
        mesh=mesh,
        compiler_params=pltpu.CompilerParams(
            use_tc_tiling_on_sc=False, needs_layout_passes=False),
        scratch_types=[
            pltpu.VMEM((BPW * SEQ,), jnp.int32),
            pltpu.VMEM((C0, PACKED), jnp.int32),
            pltpu.VMEM((C1, PACKED), jnp.int32),
            pltpu.VMEM((C0, PACKED), jnp.int32),
            pltpu.VMEM((C1, PACKED), jnp.int32),
            pltpu.VMEM((BPW, EMBED), jnp.float32),
            pltpu.SemaphoreType.DMA,
            pltpu.SemaphoreType.DMA,
        ],
    )
    def k(ids_hbm, tbl_hbm, out_hbm, idx_v, a0, a1, b0, b1, pooled_v,
          sem_a, sem_b):
        wid = lax.axis_index("s") * NC + lax.axis_index("c")
        base = wid * BPW
        pltpu.sync_copy(ids_hbm.at[pl.ds(base * SEQ, BPW * SEQ)], idx_v)

        def start(b, r0, r1, sem):
            off = b * SEQ
            pltpu.make_async_copy(
                tbl_hbm.at[idx_v.at[pl.ds(off, C0)]], r0, sem).start()
            pltpu.make_async_copy(
                tbl_hbm.at[idx_v.at[pl.ds(off + C0, C1)]], r1, sem).start()

        def drain(r0, r1, sem):
            pltpu.make_async_copy(
                tbl_hbm.at[idx_v.at[pl.ds(0, C0)]], r0, sem).wait()
            pltpu.make_async_copy(
                tbl_hbm.at[idx_v.at[pl.ds(0, C1)]], r1, sem).wait()

        def row_add(ref, r, acc):
            new = []
            for h in range(2):
                w = plsc.bitcast(
                    ref[r, pl.ds(h * LANES, LANES)], jnp.bfloat16)
                lo, hi = plsc.unpack(w, format=plsc.PackFormat.INTERLEAVED)
                new.append(acc[2 * h] + lo)
                new.append(acc[2 * h + 1] + hi)
            return tuple(new)

        def accumulate(b, r0, r1):
            acc = lax.fori_loop(
                0, C0, lambda r, acc: row_add(r0, r, acc),
                tuple(jnp.zeros((LANES,), jnp.float32) for _ in range(4)),
            )
            acc = lax.fori_loop(
                0, C1, lambda r, acc: row_add(r1, r, acc), acc)

            evens = lax.iota(jnp.int32, LANES) * 2
            for h in range(2):
                colbase = h * 2 * LANES
                plsc.store_scatter(
                    pooled_v.at[b], [colbase + evens], acc[2 * h])
                plsc.store_scatter(
                    pooled_v.at[b], [colbase + evens + 1], acc[2 * h + 1])

        start(0, a0, a1, sem_a)

        @pl.loop(0, BPW, step=2)
        def _(b):
            start(b + 1, b0, b1, sem_b)
            drain(a0, a1, sem_a)
            accumulate(b, a0, a1)

            @pl.when(b + 2 < BPW)
            def _():
                start(b + 2, a0, a1, sem_a)

            drain(b0, b1, sem_b)
            accumulate(b + 1, b0, b1)

        pltpu.sync_copy(pooled_v, out_hbm.at[pl.ds(base, BPW)])

    return k(flat_ids, table_p)


def _mlp(pooled, W1, b1, W2, b2):
    BB = 512

    def body(p_ref, w1_ref, b1_ref, w2_ref, b2_ref, o_ref):
        x = p_ref[...] * (1.0 / SEQ)
        h = jnp.dot(x, w1_ref[...], preferred_element_type=jnp.float32)
        h = jnp.maximum(h + b1_ref[...], 0.0)
        o_ref[...] = (
            jnp.dot(h, w2_ref[...], preferred_element_type=jnp.float32)
            + b2_ref[...]
        )

    return pl.pallas_call(
        body,
        grid=(BATCH // BB,),
        in_specs=[
            pl.BlockSpec((BB, EMBED), lambda i: (i, 0)),
            pl.BlockSpec((EMBED, HIDDEN), lambda i: (0, 0)),
            pl.BlockSpec((1, HIDDEN), lambda i: (0, 0)),
            pl.BlockSpec((HIDDEN, NUM_OUT), lambda i: (0, 0)),
            pl.BlockSpec((1, NUM_OUT), lambda i: (0, 0)),
        ],
        out_specs=pl.BlockSpec((BB, NUM_OUT), lambda i: (i, 0)),
        out_shape=jax.ShapeDtypeStruct((BATCH, NUM_OUT), jnp.float32),
    )(pooled, W1, b1.reshape(1, HIDDEN), W2, b2.reshape(1, NUM_OUT))


def kernel(input_ids, embedding, W1, b1, W2, b2):
    flat_ids = input_ids.reshape(-1).astype(jnp.int32)
    embT = embedding.T
    packed = _sc_transpose_pack(embT, embT[:, NTC_FULL * 128:])
    table_p = packed.reshape(VOCAB, PACKED)
    pooled = _sc_gather_pool(flat_ids, table_p)
    return _mlp(pooled, W1, b1, W2, b2)
